# baseline (speedup 1.0000x reference)
import functools

import jax
import jax.numpy as jnp
from jax import lax
from jax.experimental import pallas as pl
from jax.experimental.pallas import tpu as pltpu
from jax.experimental.pallas import tpu_sc as plsc


def _sc_pool(data, table):
    B, L = data.shape
    _, D = table.shape
    try:
        info = plsc.get_sparse_core_info()
        NC, NS = info.num_cores, info.num_subcores
    except Exception:
        NC, NS = 2, 16
    NW = NC * NS
    assert B % NW == 0 and L % 2 == 0 and D % 16 == 0
    BPW = B // NW
    C1 = -(-(L // 2) // 8) * 8
    C2 = L - C1
    CHUNKS = ((0, C1), (C1, C2))
    assert C1 <= 128 and 0 < C2 <= 128 and C2 % 8 == 0
    mesh = plsc.VectorSubcoreMesh(core_axis_name="c", subcore_axis_name="s")

    @functools.partial(
        pl.kernel,
        out_type=jax.ShapeDtypeStruct((B, D), jnp.float32),
        mesh=mesh,
        scratch_types=[
            pltpu.VMEM((BPW, L), jnp.int32),
            pltpu.VMEM((2, L, D), jnp.float32),
            pltpu.VMEM((BPW, D), jnp.float32),
            pltpu.SemaphoreType.DMA,
            pltpu.SemaphoreType.DMA,
        ],
        compiler_params=pltpu.CompilerParams(use_tc_tiling_on_sc=False),
    )
    def k(data_hbm, table_hbm, out_hbm, idx_v, rows_v, out_v, sem0, sem1):
        wid = lax.axis_index("s") * NC + lax.axis_index("c")
        base = wid * BPW
        pltpu.sync_copy(data_hbm.at[pl.ds(base, BPW), :], idx_v)
        sems = (sem0, sem1)

        def start(b, buf):
            sem = sems[buf]
            for off, n in CHUNKS:
                pltpu.make_async_copy(
                    table_hbm.at[idx_v.at[b, pl.ds(off, n)]],
                    rows_v.at[buf, pl.ds(off, n)], sem).start()

        def wait(buf):
            sem = sems[buf]
            for off, n in CHUNKS:
                pltpu.make_async_copy(
                    table_hbm.at[idx_v.at[0, pl.ds(off, n)]],
                    rows_v.at[buf, pl.ds(off, n)], sem).wait()

        def reduce(b, buf):
            z = jnp.zeros((16,), jnp.float32)

            @pl.loop(0, L, init_carry=(z,) * (D // 16), unroll=8)
            def acc(j, carry):
                return tuple(
                    c + rows_v[buf, j, pl.ds(16 * t, 16)]
                    for t, c in enumerate(carry))

            for t in range(D // 16):
                out_v[b, pl.ds(16 * t, 16)] = acc[t]

        start(0, 0)
        start(1, 1)

        @pl.loop(0, BPW // 2 - 1)
        def _(i2):
            b0 = i2 * 2
            wait(0)
            reduce(b0, 0)
            start(b0 + 2, 0)
            wait(1)
            reduce(b0 + 1, 1)
            start(b0 + 3, 1)

        wait(0)
        reduce(BPW - 2, 0)
        wait(1)
        reduce(BPW - 1, 1)
        pltpu.sync_copy(out_v, out_hbm.at[pl.ds(base, BPW), :])

    return k(data, table)


def _head_body(pooled_ref, data_ref, len_ref, t0_ref, w_ref, b_ref, out_ref):
    cnt0 = jnp.sum((data_ref[...] == 0).astype(jnp.float32), axis=1,
                   keepdims=True)
    x = (pooled_ref[...] - cnt0 * t0_ref[...]) / len_ref[...].astype(jnp.float32)
    logits = lax.dot_general(x, w_ref[...], (((1,), (1,)), ((), ())),
                             preferred_element_type=jnp.float32) + b_ref[...]
    m = jnp.max(logits, axis=1, keepdims=True)
    s = logits - m
    out_ref[...] = s - jnp.log(jnp.sum(jnp.exp(s), axis=1, keepdims=True))


def _tc_head(pooled, data, length, table0, W, b):
    B, D = pooled.shape
    L = data.shape[1]
    C = W.shape[0]
    BB = 1024
    grid = (B // BB,)
    return pl.pallas_call(
        _head_body,
        grid=grid,
        in_specs=[
            pl.BlockSpec((BB, D), lambda i: (i, 0)),
            pl.BlockSpec((BB, L), lambda i: (i, 0)),
            pl.BlockSpec((BB, 1), lambda i: (i, 0)),
            pl.BlockSpec((1, D), lambda i: (0, 0)),
            pl.BlockSpec((C, D), lambda i: (0, 0)),
            pl.BlockSpec((1, C), lambda i: (0, 0)),
        ],
        out_specs=pl.BlockSpec((BB, C), lambda i: (i, 0)),
        out_shape=jax.ShapeDtypeStruct((B, C), jnp.float32),
    )(pooled, data, length.reshape(B, 1), table0, W, b.reshape(1, C))


def kernel(data, length, table, W, b):
    pooled = _sc_pool(data, table)
    t0 = lax.slice(table, (0, 0), (1, table.shape[1]))
    return _tc_head(pooled, data, length, t0, W, b)


# speedup vs baseline: 1.0631x; 1.0631x over previous
import functools

import jax
import jax.numpy as jnp
from jax import lax
from jax.experimental import pallas as pl
from jax.experimental.pallas import tpu as pltpu
from jax.experimental.pallas import tpu_sc as plsc


def _sc_pool(data_flat, table, B, L, D):
    _, DP = table.shape
    try:
        info = plsc.get_sparse_core_info()
        NC, NS = info.num_cores, info.num_subcores
    except Exception:
        NC, NS = 2, 16
    NW = NC * NS
    assert B % NW == 0 and L % 2 == 0 and D % 16 == 0
    BPW = B // NW
    C1 = -(-(L // 2) // 8) * 8
    C2 = L - C1
    CHUNKS = ((0, C1), (C1, C2))
    assert C1 <= 128 and 0 < C2 <= 128 and C2 % 8 == 0
    mesh = plsc.VectorSubcoreMesh(core_axis_name="c", subcore_axis_name="s")

    @functools.partial(
        pl.kernel,
        out_type=jax.ShapeDtypeStruct((B, D), jnp.float32),
        mesh=mesh,
        scratch_types=[
            pltpu.VMEM((BPW * L,), jnp.int32),
            pltpu.VMEM((2, L, DP), jnp.float32),
            pltpu.VMEM((BPW, D), jnp.float32),
            pltpu.SemaphoreType.DMA,
            pltpu.SemaphoreType.DMA,
        ],
    )
    def k(data_hbm, table_hbm, out_hbm, idx_v, rows_v, out_v, sem0, sem1):
        wid = lax.axis_index("s") * NC + lax.axis_index("c")
        base = wid * BPW
        pltpu.sync_copy(data_hbm.at[pl.ds(base * L, BPW * L)], idx_v)
        sems = (sem0, sem1)

        def start(b, buf):
            sem = sems[buf]
            for off, n in CHUNKS:
                pltpu.make_async_copy(
                    table_hbm.at[idx_v.at[pl.ds(b * L + off, n)]],
                    rows_v.at[buf, pl.ds(off, n)], sem).start()

        def wait(buf):
            sem = sems[buf]
            for off, n in CHUNKS:
                pltpu.make_async_copy(
                    table_hbm.at[idx_v.at[pl.ds(off, n)]],
                    rows_v.at[buf, pl.ds(off, n)], sem).wait()

        def reduce(b, buf):
            z = jnp.zeros((16,), jnp.float32)

            @pl.loop(0, L, init_carry=(z,) * (D // 16), unroll=8)
            def acc(j, carry):
                return tuple(
                    c + rows_v[buf, j, pl.ds(16 * t, 16)]
                    for t, c in enumerate(carry))

            for t in range(D // 16):
                out_v[b, pl.ds(16 * t, 16)] = acc[t]

        start(0, 0)
        start(1, 1)

        @pl.loop(0, BPW // 2 - 1)
        def _(i2):
            b0 = i2 * 2
            wait(0)
            reduce(b0, 0)
            start(b0 + 2, 0)
            wait(1)
            reduce(b0 + 1, 1)
            start(b0 + 3, 1)

        wait(0)
        reduce(BPW - 2, 0)
        wait(1)
        reduce(BPW - 1, 1)
        pltpu.sync_copy(out_v, out_hbm.at[pl.ds(base, BPW), :])

    return k(data_flat, table)


def _tp_body(tT_ref, out_ref):
    x = tT_ref[...]
    y = jnp.swapaxes(x, 0, 1)
    pad = jnp.zeros((y.shape[0], 128 - y.shape[1]), jnp.float32)
    out_ref[...] = jnp.concatenate([y, pad], axis=1)


def _tc_transpose_pad(tableT):
    D, V = tableT.shape
    VB = 2048
    return pl.pallas_call(
        _tp_body,
        grid=(pl.cdiv(V, VB),),
        in_specs=[pl.BlockSpec((D, VB), lambda i: (0, i))],
        out_specs=pl.BlockSpec((VB, 128), lambda i: (i, 0)),
        out_shape=jax.ShapeDtypeStruct((V, 128), jnp.float32),
    )(tableT)


def _head_body(pooled_ref, data_ref, len_ref, t0_ref, w_ref, b_ref, out_ref):
    cnt0 = jnp.sum((data_ref[...] == 0).astype(jnp.float32), axis=1,
                   keepdims=True)
    x = (pooled_ref[...] - cnt0 * t0_ref[...]) / len_ref[...].astype(jnp.float32)
    logits = lax.dot_general(x, w_ref[...], (((1,), (1,)), ((), ())),
                             preferred_element_type=jnp.float32) + b_ref[...]
    m = jnp.max(logits, axis=1, keepdims=True)
    s = logits - m
    out_ref[...] = s - jnp.log(jnp.sum(jnp.exp(s), axis=1, keepdims=True))


def _tc_head(pooled, data, length, table0, W, b):
    B, D = pooled.shape
    L = data.shape[1]
    C = W.shape[0]
    BB = 1024
    grid = (B // BB,)
    return pl.pallas_call(
        _head_body,
        grid=grid,
        in_specs=[
            pl.BlockSpec((BB, D), lambda i: (i, 0)),
            pl.BlockSpec((BB, L), lambda i: (i, 0)),
            pl.BlockSpec((BB, 1), lambda i: (i, 0)),
            pl.BlockSpec((1, D), lambda i: (0, 0)),
            pl.BlockSpec((C, D), lambda i: (0, 0)),
            pl.BlockSpec((1, C), lambda i: (0, 0)),
        ],
        out_specs=pl.BlockSpec((BB, C), lambda i: (i, 0)),
        out_shape=jax.ShapeDtypeStruct((B, C), jnp.float32),
    )(pooled, data, length.reshape(B, 1), table0, W, b.reshape(1, C))


def kernel(data, length, table, W, b):
    B, L = data.shape
    D = table.shape[1]
    tableP = _tc_transpose_pad(table.T)
    pooled = _sc_pool(data.reshape(B * L), tableP, B, L, D)
    t0 = lax.slice(table, (0, 0), (1, D))
    return _tc_head(pooled, data, length, t0, W, b)


# speedup vs baseline: 1.2197x; 1.1473x over previous
import functools

import jax
import jax.numpy as jnp
from jax import lax
from jax.experimental import pallas as pl
from jax.experimental.pallas import tpu as pltpu
from jax.experimental.pallas import tpu_sc as plsc

_VB = 2048
_HF = _VB // 2


def _tb_body(tT_ref, out_ref):
    x = tT_ref[...]
    z = jnp.swapaxes(x, 0, 1)
    out_ref[...] = jnp.concatenate([z[:_HF], z[_HF:]], axis=1)


def _tc_build_table(tableT):
    D, V = tableT.shape
    nblk = pl.cdiv(V, _VB)
    return pl.pallas_call(
        _tb_body,
        grid=(nblk,),
        in_specs=[pl.BlockSpec((D, _VB), lambda i: (0, i))],
        out_specs=pl.BlockSpec((_HF, 2 * D), lambda i: (i, 0)),
        out_shape=jax.ShapeDtypeStruct((nblk * _HF, 2 * D), jnp.float32),
    )(tableT)


def _sc_pool(data_flat, tableR, B, L):
    _, D = tableR.shape
    try:
        info = plsc.get_sparse_core_info()
        NC, NS = info.num_cores, info.num_subcores
    except Exception:
        NC, NS = 2, 16
    NW = NC * NS
    assert B % NW == 0 and L % 2 == 0 and D % 16 == 0
    BPW = B // NW
    C1 = -(-(L // 2) // 8) * 8
    C2 = L - C1
    CHUNKS = ((0, C1), (C1, C2))
    assert C1 <= 128 and 0 < C2 <= 128 and C2 % 8 == 0
    mesh = plsc.VectorSubcoreMesh(core_axis_name="c", subcore_axis_name="s")

    @functools.partial(
        pl.kernel,
        out_type=jax.ShapeDtypeStruct((B, D), jnp.float32),
        mesh=mesh,
        scratch_types=[
            pltpu.VMEM((BPW * L,), jnp.int32),
            pltpu.VMEM((2, L, D), jnp.float32),
            pltpu.VMEM((BPW, D), jnp.float32),
            pltpu.SemaphoreType.DMA,
            pltpu.SemaphoreType.DMA,
        ],
        compiler_params=pltpu.CompilerParams(use_tc_tiling_on_sc=False),
    )
    def k(data_hbm, table_hbm, out_hbm, idx_v, rows_v, out_v, sem0, sem1):
        wid = lax.axis_index("s") * NC + lax.axis_index("c")
        base = wid * BPW
        pltpu.sync_copy(data_hbm.at[pl.ds(base * L, BPW * L)], idx_v)
        sems = (sem0, sem1)

        @pl.loop(0, BPW * L // 16)
        def _(g):
            r = idx_v[pl.ds(g * 16, 16)]
            q = ((r >> 11) << 11) | ((r & (_HF - 1)) << 1) | ((r >> 10) & 1)
            idx_v[pl.ds(g * 16, 16)] = q

        def start(b, buf):
            sem = sems[buf]
            for off, n in CHUNKS:
                pltpu.make_async_copy(
                    table_hbm.at[idx_v.at[pl.ds(b * L + off, n)]],
                    rows_v.at[buf, pl.ds(off, n)], sem).start()

        def wait(buf):
            sem = sems[buf]
            for off, n in CHUNKS:
                pltpu.make_async_copy(
                    table_hbm.at[idx_v.at[pl.ds(off, n)]],
                    rows_v.at[buf, pl.ds(off, n)], sem).wait()

        def reduce(b, buf):
            z = jnp.zeros((16,), jnp.float32)

            @pl.loop(0, L, init_carry=(z,) * (D // 16), unroll=8)
            def acc(j, carry):
                return tuple(
                    c + rows_v[buf, j, pl.ds(16 * t, 16)]
                    for t, c in enumerate(carry))

            for t in range(D // 16):
                out_v[b, pl.ds(16 * t, 16)] = acc[t]

        start(0, 0)
        start(1, 1)

        @pl.loop(0, BPW // 2 - 1)
        def _(i2):
            b0 = i2 * 2
            wait(0)
            reduce(b0, 0)
            start(b0 + 2, 0)
            wait(1)
            reduce(b0 + 1, 1)
            start(b0 + 3, 1)

        wait(0)
        reduce(BPW - 2, 0)
        wait(1)
        reduce(BPW - 1, 1)
        pltpu.sync_copy(out_v, out_hbm.at[pl.ds(base, BPW), :])

    return k(data_flat, tableR)


def _head_body(pooled_ref, data_ref, len_ref, t0_ref, w_ref, b_ref, out_ref):
    cnt0 = jnp.sum((data_ref[...] == 0).astype(jnp.float32), axis=1,
                   keepdims=True)
    x = (pooled_ref[...] - cnt0 * t0_ref[...]) / len_ref[...].astype(jnp.float32)
    logits = lax.dot_general(x, w_ref[...], (((1,), (1,)), ((), ())),
                             preferred_element_type=jnp.float32) + b_ref[...]
    m = jnp.max(logits, axis=1, keepdims=True)
    s = logits - m
    out_ref[...] = s - jnp.log(jnp.sum(jnp.exp(s), axis=1, keepdims=True))


def _tc_head(pooled, data, length, table0, W, b):
    B, D = pooled.shape
    L = data.shape[1]
    C = W.shape[0]
    BB = 1024
    grid = (B // BB,)
    return pl.pallas_call(
        _head_body,
        grid=grid,
        in_specs=[
            pl.BlockSpec((BB, D), lambda i: (i, 0)),
            pl.BlockSpec((BB, L), lambda i: (i, 0)),
            pl.BlockSpec((BB, 1), lambda i: (i, 0)),
            pl.BlockSpec((1, D), lambda i: (0, 0)),
            pl.BlockSpec((C, D), lambda i: (0, 0)),
            pl.BlockSpec((1, C), lambda i: (0, 0)),
        ],
        out_specs=pl.BlockSpec((BB, C), lambda i: (i, 0)),
        out_shape=jax.ShapeDtypeStruct((B, C), jnp.float32),
    )(pooled, data, length.reshape(B, 1), table0, W, b.reshape(1, C))


def kernel(data, length, table, W, b):
    B, L = data.shape
    D = table.shape[1]
    tableM = _tc_build_table(table.T)
    tableR = tableM.reshape(-1, D)
    pooled = _sc_pool(data.reshape(B * L), tableR, B, L)
    t0 = lax.slice(table, (0, 0), (1, D))
    return _tc_head(pooled, data, length, t0, W, b)


# speedup vs baseline: 1.7797x; 1.4591x over previous
import functools

import jax
import jax.numpy as jnp
from jax import lax
from jax.experimental import pallas as pl
from jax.experimental.pallas import tpu as pltpu
from jax.experimental.pallas import tpu_sc as plsc

_VB = 8192
_HF = _VB // 2
_SB = _VB.bit_length() - 1
_SH = _SB - 1


def _tb_body(tT_ref, out_ref):
    x = tT_ref[...]
    z = jnp.swapaxes(x, 0, 1)
    out_ref[...] = jnp.concatenate([z[:_HF], z[_HF:]], axis=1)


def _tc_build_table(tableT):
    D, V = tableT.shape
    nblk = pl.cdiv(V, _VB)
    return pl.pallas_call(
        _tb_body,
        grid=(nblk,),
        in_specs=[pl.BlockSpec((D, _VB), lambda i: (0, i))],
        out_specs=pl.BlockSpec((_HF, 2 * D), lambda i: (i, 0)),
        out_shape=jax.ShapeDtypeStruct((nblk * _HF, 2 * D), jnp.float32),
    )(tableT)


def _sc_pool(data_flat, tableR, B, L):
    _, D = tableR.shape
    try:
        info = plsc.get_sparse_core_info()
        NC, NS = info.num_cores, info.num_subcores
    except Exception:
        NC, NS = 2, 16
    NW = NC * NS
    assert B % NW == 0 and L % 2 == 0 and D % 16 == 0
    BPW = B // NW
    C1 = -(-(L // 2) // 8) * 8
    C2 = L - C1
    CHUNKS = ((0, C1), (C1, C2))
    assert C1 <= 128 and 0 < C2 <= 128 and C2 % 8 == 0
    mesh = plsc.VectorSubcoreMesh(core_axis_name="c", subcore_axis_name="s")

    @functools.partial(
        pl.kernel,
        out_type=jax.ShapeDtypeStruct((B, D), jnp.float32),
        mesh=mesh,
        scratch_types=[
            pltpu.VMEM((BPW * L,), jnp.int32),
            pltpu.VMEM((2, L, D), jnp.float32),
            pltpu.VMEM((BPW, D), jnp.float32),
            pltpu.SemaphoreType.DMA,
            pltpu.SemaphoreType.DMA,
        ],
        compiler_params=pltpu.CompilerParams(use_tc_tiling_on_sc=False),
    )
    def k(data_hbm, table_hbm, out_hbm, idx_v, rows_v, out_v, sem0, sem1):
        wid = lax.axis_index("s") * NC + lax.axis_index("c")
        base = wid * BPW
        pltpu.sync_copy(data_hbm.at[pl.ds(base * L, BPW * L)], idx_v)
        sems = (sem0, sem1)

        @pl.loop(0, BPW * L // 16)
        def _(g):
            r = idx_v[pl.ds(g * 16, 16)]
            q = ((r >> _SB) << _SB) | ((r & (_HF - 1)) << 1) | ((r >> _SH) & 1)
            idx_v[pl.ds(g * 16, 16)] = q

        def start(b, buf):
            sem = sems[buf]
            for off, n in CHUNKS:
                pltpu.make_async_copy(
                    table_hbm.at[idx_v.at[pl.ds(b * L + off, n)]],
                    rows_v.at[buf, pl.ds(off, n)], sem).start()

        def wait(buf):
            sem = sems[buf]
            for off, n in CHUNKS:
                pltpu.make_async_copy(
                    table_hbm.at[idx_v.at[pl.ds(off, n)]],
                    rows_v.at[buf, pl.ds(off, n)], sem).wait()

        def reduce(b, buf):
            z = jnp.zeros((16,), jnp.float32)

            @pl.loop(0, L, init_carry=(z,) * (D // 16), unroll=8)
            def acc(j, carry):
                return tuple(
                    c + rows_v[buf, j, pl.ds(16 * t, 16)]
                    for t, c in enumerate(carry))

            for t in range(D // 16):
                out_v[b, pl.ds(16 * t, 16)] = acc[t]

        start(0, 0)
        start(1, 1)

        @pl.loop(0, BPW // 2 - 1)
        def _(i2):
            b0 = i2 * 2
            wait(0)
            reduce(b0, 0)
            start(b0 + 2, 0)
            wait(1)
            reduce(b0 + 1, 1)
            start(b0 + 3, 1)

        wait(0)
        reduce(BPW - 2, 0)
        wait(1)
        reduce(BPW - 1, 1)
        pltpu.sync_copy(out_v, out_hbm.at[pl.ds(base, BPW), :])

    return k(data_flat, tableR)


def _head_body(pooled_ref, data_ref, len_ref, t0_ref, w_ref, b_ref, out_ref):
    cnt0 = jnp.sum((data_ref[...] == 0).astype(jnp.float32), axis=1,
                   keepdims=True)
    x = (pooled_ref[...] - cnt0 * t0_ref[...]) / len_ref[...].astype(jnp.float32)
    logits = lax.dot_general(x, w_ref[...], (((1,), (1,)), ((), ())),
                             preferred_element_type=jnp.float32) + b_ref[...]
    m = jnp.max(logits, axis=1, keepdims=True)
    s = logits - m
    out_ref[...] = s - jnp.log(jnp.sum(jnp.exp(s), axis=1, keepdims=True))


def _tc_head(pooled, data, length, table0, W, b):
    B, D = pooled.shape
    L = data.shape[1]
    C = W.shape[0]
    BB = 1024
    grid = (B // BB,)
    return pl.pallas_call(
        _head_body,
        grid=grid,
        in_specs=[
            pl.BlockSpec((BB, D), lambda i: (i, 0)),
            pl.BlockSpec((BB, L), lambda i: (i, 0)),
            pl.BlockSpec((BB, 1), lambda i: (i, 0)),
            pl.BlockSpec((1, D), lambda i: (0, 0)),
            pl.BlockSpec((C, D), lambda i: (0, 0)),
            pl.BlockSpec((1, C), lambda i: (0, 0)),
        ],
        out_specs=pl.BlockSpec((BB, C), lambda i: (i, 0)),
        out_shape=jax.ShapeDtypeStruct((B, C), jnp.float32),
    )(pooled, data, length.reshape(B, 1), table0, W, b.reshape(1, C))


def kernel(data, length, table, W, b):
    B, L = data.shape
    D = table.shape[1]
    tableM = _tc_build_table(table.T)
    tableR = tableM.reshape(-1, D)
    pooled = _sc_pool(data.reshape(B * L), tableR, B, L)
    t0 = lax.slice(table, (0, 0), (1, D))
    return _tc_head(pooled, data, length, t0, W, b)


# speedup vs baseline: 1.9196x; 1.0786x over previous
import functools

import jax
import jax.numpy as jnp
from jax import lax
from jax.experimental import pallas as pl
from jax.experimental.pallas import tpu as pltpu
from jax.experimental.pallas import tpu_sc as plsc

_VB = 16384
_HF = _VB // 2
_SB = _VB.bit_length() - 1
_SH = _SB - 1


def _tb_body(tT_ref, out_ref):
    x = tT_ref[...]
    z = jnp.swapaxes(x, 0, 1)
    out_ref[...] = jnp.concatenate([z[:_HF], z[_HF:]], axis=1)


def _tc_build_table(tableT):
    D, V = tableT.shape
    nblk = pl.cdiv(V, _VB)
    return pl.pallas_call(
        _tb_body,
        grid=(nblk,),
        in_specs=[pl.BlockSpec((D, _VB), lambda i: (0, i))],
        out_specs=pl.BlockSpec((_HF, 2 * D), lambda i: (i, 0)),
        out_shape=jax.ShapeDtypeStruct((nblk * _HF, 2 * D), jnp.float32),
    )(tableT)


def _sc_pool(data_flat, tableR, B, L):
    _, D = tableR.shape
    try:
        info = plsc.get_sparse_core_info()
        NC, NS = info.num_cores, info.num_subcores
    except Exception:
        NC, NS = 2, 16
    NW = NC * NS
    assert B % NW == 0 and L % 2 == 0 and D % 16 == 0
    BPW = B // NW
    C1 = -(-(L // 2) // 8) * 8
    C2 = L - C1
    CHUNKS = ((0, C1), (C1, C2))
    assert C1 <= 128 and 0 < C2 <= 128 and C2 % 8 == 0
    mesh = plsc.VectorSubcoreMesh(core_axis_name="c", subcore_axis_name="s")

    @functools.partial(
        pl.kernel,
        out_type=jax.ShapeDtypeStruct((B, D), jnp.float32),
        mesh=mesh,
        scratch_types=[
            pltpu.VMEM((BPW * L,), jnp.int32),
            pltpu.VMEM((2, L, D), jnp.float32),
            pltpu.VMEM((BPW, D), jnp.float32),
            pltpu.SemaphoreType.DMA,
            pltpu.SemaphoreType.DMA,
        ],
        compiler_params=pltpu.CompilerParams(use_tc_tiling_on_sc=False),
    )
    def k(data_hbm, table_hbm, out_hbm, idx_v, rows_v, out_v, sem0, sem1):
        wid = lax.axis_index("s") * NC + lax.axis_index("c")
        base = wid * BPW
        pltpu.sync_copy(data_hbm.at[pl.ds(base * L, BPW * L)], idx_v)
        sems = (sem0, sem1)

        @pl.loop(0, BPW * L // 16)
        def _(g):
            r = idx_v[pl.ds(g * 16, 16)]
            q = ((r >> _SB) << _SB) | ((r & (_HF - 1)) << 1) | ((r >> _SH) & 1)
            idx_v[pl.ds(g * 16, 16)] = q

        def start(b, buf):
            sem = sems[buf]
            for off, n in CHUNKS:
                pltpu.make_async_copy(
                    table_hbm.at[idx_v.at[pl.ds(b * L + off, n)]],
                    rows_v.at[buf, pl.ds(off, n)], sem).start()

        def wait(buf):
            sem = sems[buf]
            for off, n in CHUNKS:
                pltpu.make_async_copy(
                    table_hbm.at[idx_v.at[pl.ds(off, n)]],
                    rows_v.at[buf, pl.ds(off, n)], sem).wait()

        def reduce(b, buf):
            z = jnp.zeros((16,), jnp.float32)

            @pl.loop(0, L, init_carry=(z,) * (D // 16), unroll=8)
            def acc(j, carry):
                return tuple(
                    c + rows_v[buf, j, pl.ds(16 * t, 16)]
                    for t, c in enumerate(carry))

            for t in range(D // 16):
                out_v[b, pl.ds(16 * t, 16)] = acc[t]

        start(0, 0)
        start(1, 1)

        @pl.loop(0, BPW // 2 - 1)
        def _(i2):
            b0 = i2 * 2
            wait(0)
            reduce(b0, 0)
            start(b0 + 2, 0)
            wait(1)
            reduce(b0 + 1, 1)
            start(b0 + 3, 1)

        wait(0)
        reduce(BPW - 2, 0)
        wait(1)
        reduce(BPW - 1, 1)
        pltpu.sync_copy(out_v, out_hbm.at[pl.ds(base, BPW), :])

    return k(data_flat, tableR)


def _head_body(pooled_ref, data_ref, len_ref, t0_ref, w_ref, b_ref, out_ref):
    cnt0 = jnp.sum((data_ref[...] == 0).astype(jnp.float32), axis=1,
                   keepdims=True)
    x = (pooled_ref[...] - cnt0 * t0_ref[...]) / len_ref[...].astype(jnp.float32)
    logits = lax.dot_general(x, w_ref[...], (((1,), (1,)), ((), ())),
                             preferred_element_type=jnp.float32) + b_ref[...]
    m = jnp.max(logits, axis=1, keepdims=True)
    s = logits - m
    out_ref[...] = s - jnp.log(jnp.sum(jnp.exp(s), axis=1, keepdims=True))


def _tc_head(pooled, data, length, table0, W, b):
    B, D = pooled.shape
    L = data.shape[1]
    C = W.shape[0]
    BB = 1024
    grid = (B // BB,)
    return pl.pallas_call(
        _head_body,
        grid=grid,
        in_specs=[
            pl.BlockSpec((BB, D), lambda i: (i, 0)),
            pl.BlockSpec((BB, L), lambda i: (i, 0)),
            pl.BlockSpec((BB, 1), lambda i: (i, 0)),
            pl.BlockSpec((1, D), lambda i: (0, 0)),
            pl.BlockSpec((C, D), lambda i: (0, 0)),
            pl.BlockSpec((1, C), lambda i: (0, 0)),
        ],
        out_specs=pl.BlockSpec((BB, C), lambda i: (i, 0)),
        out_shape=jax.ShapeDtypeStruct((B, C), jnp.float32),
    )(pooled, data, length.reshape(B, 1), table0, W, b.reshape(1, C))


def kernel(data, length, table, W, b):
    B, L = data.shape
    D = table.shape[1]
    tableM = _tc_build_table(table.T)
    tableR = tableM.reshape(-1, D)
    pooled = _sc_pool(data.reshape(B * L), tableR, B, L)
    t0 = lax.slice(table, (0, 0), (1, D))
    return _tc_head(pooled, data, length, t0, W, b)


# speedup vs baseline: 2.0066x; 1.0453x over previous
import functools

import jax
import jax.numpy as jnp
from jax import lax
from jax.experimental import pallas as pl
from jax.experimental.pallas import tpu as pltpu
from jax.experimental.pallas import tpu_sc as plsc

_VB = 32768
_HF = _VB // 2
_SB = _VB.bit_length() - 1
_SH = _SB - 1


def _tb_body(tT_ref, out_ref):
    x = tT_ref[...]
    z = jnp.swapaxes(x, 0, 1)
    out_ref[...] = jnp.concatenate([z[:_HF], z[_HF:]], axis=1)


def _tc_build_table(tableT):
    D, V = tableT.shape
    nblk = pl.cdiv(V, _VB)
    return pl.pallas_call(
        _tb_body,
        grid=(nblk,),
        in_specs=[pl.BlockSpec((D, _VB), lambda i: (0, i))],
        out_specs=pl.BlockSpec((_HF, 2 * D), lambda i: (i, 0)),
        out_shape=jax.ShapeDtypeStruct((nblk * _HF, 2 * D), jnp.float32),
    )(tableT)


def _sc_pool(data_flat, tableR, B, L):
    _, D = tableR.shape
    try:
        info = plsc.get_sparse_core_info()
        NC, NS = info.num_cores, info.num_subcores
    except Exception:
        NC, NS = 2, 16
    NW = NC * NS
    assert B % NW == 0 and L % 2 == 0 and D % 16 == 0
    BPW = B // NW
    C1 = -(-(L // 2) // 8) * 8
    C2 = L - C1
    CHUNKS = ((0, C1), (C1, C2))
    assert C1 <= 128 and 0 < C2 <= 128 and C2 % 8 == 0
    mesh = plsc.VectorSubcoreMesh(core_axis_name="c", subcore_axis_name="s")

    @functools.partial(
        pl.kernel,
        out_type=jax.ShapeDtypeStruct((B, D), jnp.float32),
        mesh=mesh,
        scratch_types=[
            pltpu.VMEM((BPW * L,), jnp.int32),
            pltpu.VMEM((2, L, D), jnp.float32),
            pltpu.VMEM((BPW, D), jnp.float32),
            pltpu.SemaphoreType.DMA,
            pltpu.SemaphoreType.DMA,
        ],
        compiler_params=pltpu.CompilerParams(use_tc_tiling_on_sc=False),
    )
    def k(data_hbm, table_hbm, out_hbm, idx_v, rows_v, out_v, sem0, sem1):
        wid = lax.axis_index("s") * NC + lax.axis_index("c")
        base = wid * BPW
        pltpu.sync_copy(data_hbm.at[pl.ds(base * L, BPW * L)], idx_v)
        sems = (sem0, sem1)

        @pl.loop(0, BPW * L // 16)
        def _(g):
            r = idx_v[pl.ds(g * 16, 16)]
            q = ((r >> _SB) << _SB) | ((r & (_HF - 1)) << 1) | ((r >> _SH) & 1)
            idx_v[pl.ds(g * 16, 16)] = q

        def start(b, buf):
            sem = sems[buf]
            for off, n in CHUNKS:
                pltpu.make_async_copy(
                    table_hbm.at[idx_v.at[pl.ds(b * L + off, n)]],
                    rows_v.at[buf, pl.ds(off, n)], sem).start()

        def wait(buf):
            sem = sems[buf]
            for off, n in CHUNKS:
                pltpu.make_async_copy(
                    table_hbm.at[idx_v.at[pl.ds(off, n)]],
                    rows_v.at[buf, pl.ds(off, n)], sem).wait()

        def reduce(b, buf):
            z = jnp.zeros((16,), jnp.float32)

            @pl.loop(0, L, init_carry=(z,) * (D // 16), unroll=8)
            def acc(j, carry):
                return tuple(
                    c + rows_v[buf, j, pl.ds(16 * t, 16)]
                    for t, c in enumerate(carry))

            for t in range(D // 16):
                out_v[b, pl.ds(16 * t, 16)] = acc[t]

        start(0, 0)
        start(1, 1)

        @pl.loop(0, BPW // 2 - 1)
        def _(i2):
            b0 = i2 * 2
            wait(0)
            reduce(b0, 0)
            start(b0 + 2, 0)
            wait(1)
            reduce(b0 + 1, 1)
            start(b0 + 3, 1)

        wait(0)
        reduce(BPW - 2, 0)
        wait(1)
        reduce(BPW - 1, 1)
        pltpu.sync_copy(out_v, out_hbm.at[pl.ds(base, BPW), :])

    return k(data_flat, tableR)


def _head_body(pooled_ref, data_ref, len_ref, t0_ref, w_ref, b_ref, out_ref):
    cnt0 = jnp.sum((data_ref[...] == 0).astype(jnp.float32), axis=1,
                   keepdims=True)
    x = (pooled_ref[...] - cnt0 * t0_ref[...]) / len_ref[...].astype(jnp.float32)
    logits = lax.dot_general(x, w_ref[...], (((1,), (1,)), ((), ())),
                             preferred_element_type=jnp.float32) + b_ref[...]
    m = jnp.max(logits, axis=1, keepdims=True)
    s = logits - m
    out_ref[...] = s - jnp.log(jnp.sum(jnp.exp(s), axis=1, keepdims=True))


def _tc_head(pooled, data, length, table0, W, b):
    B, D = pooled.shape
    L = data.shape[1]
    C = W.shape[0]
    BB = 1024
    grid = (B // BB,)
    return pl.pallas_call(
        _head_body,
        grid=grid,
        in_specs=[
            pl.BlockSpec((BB, D), lambda i: (i, 0)),
            pl.BlockSpec((BB, L), lambda i: (i, 0)),
            pl.BlockSpec((BB, 1), lambda i: (i, 0)),
            pl.BlockSpec((1, D), lambda i: (0, 0)),
            pl.BlockSpec((C, D), lambda i: (0, 0)),
            pl.BlockSpec((1, C), lambda i: (0, 0)),
        ],
        out_specs=pl.BlockSpec((BB, C), lambda i: (i, 0)),
        out_shape=jax.ShapeDtypeStruct((B, C), jnp.float32),
    )(pooled, data, length.reshape(B, 1), table0, W, b.reshape(1, C))


def kernel(data, length, table, W, b):
    B, L = data.shape
    D = table.shape[1]
    tableM = _tc_build_table(table.T)
    tableR = tableM.reshape(-1, D)
    pooled = _sc_pool(data.reshape(B * L), tableR, B, L)
    t0 = lax.slice(table, (0, 0), (1, D))
    return _tc_head(pooled, data, length, t0, W, b)


# speedup vs baseline: 2.0282x; 1.0108x over previous
import functools

import jax
import jax.numpy as jnp
from jax import lax
from jax.experimental import pallas as pl
from jax.experimental.pallas import tpu as pltpu
from jax.experimental.pallas import tpu_sc as plsc

_VB = 32768
_HF = _VB // 2
_SB = _VB.bit_length() - 1
_SH = _SB - 1


def _tb_body(tT_ref, out_ref):
    x = tT_ref[...]
    z = jnp.swapaxes(x, 0, 1)
    out_ref[...] = jnp.concatenate([z[:_HF], z[_HF:]], axis=1)


def _tc_build_table(tableT):
    D, V = tableT.shape
    nblk = pl.cdiv(V, _VB)
    return pl.pallas_call(
        _tb_body,
        grid=(nblk,),
        in_specs=[pl.BlockSpec((D, _VB), lambda i: (0, i))],
        out_specs=pl.BlockSpec((_HF, 2 * D), lambda i: (i, 0)),
        out_shape=jax.ShapeDtypeStruct((nblk * _HF, 2 * D), jnp.float32),
    )(tableT)


def _sc_pool(data_flat, tableR, B, L):
    _, D = tableR.shape
    try:
        info = plsc.get_sparse_core_info()
        NC, NS = info.num_cores, info.num_subcores
    except Exception:
        NC, NS = 2, 16
    NW = NC * NS
    assert B % NW == 0 and L % 2 == 0 and D % 16 == 0
    BPW = B // NW
    M = 16
    CHUNKS = [(p * M, M) for p in range(L // M)]
    if L % M:
        CHUNKS.append(((L // M) * M, L % M))
    assert all(n % 8 == 0 and off % 8 == 0 for off, n in CHUNKS)
    mesh = plsc.VectorSubcoreMesh(core_axis_name="c", subcore_axis_name="s")

    @functools.partial(
        pl.kernel,
        out_type=jax.ShapeDtypeStruct((B, D), jnp.float32),
        mesh=mesh,
        scratch_types=[
            pltpu.VMEM((BPW * L,), jnp.int32),
            pltpu.VMEM((2, M, D), jnp.float32),
            pltpu.VMEM((BPW, D), jnp.float32),
            pltpu.SemaphoreType.DMA,
            pltpu.SemaphoreType.DMA,
        ],
        compiler_params=pltpu.CompilerParams(use_tc_tiling_on_sc=False),
    )
    def k(data_hbm, table_hbm, out_hbm, idx_v, rows_v, out_v, sem0, sem1):
        wid = lax.axis_index("s") * NC + lax.axis_index("c")
        base = wid * BPW
        pltpu.sync_copy(data_hbm.at[pl.ds(base * L, BPW * L)], idx_v)
        sems = (sem0, sem1)

        @pl.loop(0, BPW * L // 16)
        def _(g):
            r = idx_v[pl.ds(g * 16, 16)]
            q = ((r >> _SB) << _SB) | ((r & (_HF - 1)) << 1) | ((r >> _SH) & 1)
            idx_v[pl.ds(g * 16, 16)] = q

        zero16 = jnp.zeros((16,), jnp.float32)

        def zero(buf):
            for m in range(M):
                for t in range(D // 16):
                    rows_v[buf, m, pl.ds(16 * t, 16)] = zero16

        def start(b, buf):
            sem = sems[buf]
            for off, n in CHUNKS:
                pltpu.async_copy(
                    table_hbm.at[idx_v.at[pl.ds(b * L + off, n)]],
                    rows_v.at[buf, pl.ds(0, n)], sem, add=True)

        def wait(buf):
            sem = sems[buf]
            for off, n in CHUNKS:
                pltpu.make_async_copy(
                    table_hbm.at[idx_v.at[pl.ds(off, n)]],
                    rows_v.at[buf, pl.ds(0, n)], sem).wait()

        def reduce(b, buf):
            for t in range(D // 16):
                a = rows_v[buf, 0, pl.ds(16 * t, 16)]
                for m in range(1, M):
                    a = a + rows_v[buf, m, pl.ds(16 * t, 16)]
                out_v[b, pl.ds(16 * t, 16)] = a

        zero(0)
        zero(1)
        start(0, 0)
        start(1, 1)

        @pl.loop(0, BPW // 2 - 1)
        def _(i2):
            b0 = i2 * 2
            wait(0)
            reduce(b0, 0)
            zero(0)
            start(b0 + 2, 0)
            wait(1)
            reduce(b0 + 1, 1)
            zero(1)
            start(b0 + 3, 1)

        wait(0)
        reduce(BPW - 2, 0)
        wait(1)
        reduce(BPW - 1, 1)
        pltpu.sync_copy(out_v, out_hbm.at[pl.ds(base, BPW), :])

    return k(data_flat, tableR)


def _head_body(pooled_ref, data_ref, len_ref, t0_ref, w_ref, b_ref, out_ref):
    cnt0 = jnp.sum((data_ref[...] == 0).astype(jnp.float32), axis=1,
                   keepdims=True)
    x = (pooled_ref[...] - cnt0 * t0_ref[...]) / len_ref[...].astype(jnp.float32)
    logits = lax.dot_general(x, w_ref[...], (((1,), (1,)), ((), ())),
                             preferred_element_type=jnp.float32) + b_ref[...]
    m = jnp.max(logits, axis=1, keepdims=True)
    s = logits - m
    out_ref[...] = s - jnp.log(jnp.sum(jnp.exp(s), axis=1, keepdims=True))


def _tc_head(pooled, data, length, table0, W, b):
    B, D = pooled.shape
    L = data.shape[1]
    C = W.shape[0]
    BB = 1024
    grid = (B // BB,)
    return pl.pallas_call(
        _head_body,
        grid=grid,
        in_specs=[
            pl.BlockSpec((BB, D), lambda i: (i, 0)),
            pl.BlockSpec((BB, L), lambda i: (i, 0)),
            pl.BlockSpec((BB, 1), lambda i: (i, 0)),
            pl.BlockSpec((1, D), lambda i: (0, 0)),
            pl.BlockSpec((C, D), lambda i: (0, 0)),
            pl.BlockSpec((1, C), lambda i: (0, 0)),
        ],
        out_specs=pl.BlockSpec((BB, C), lambda i: (i, 0)),
        out_shape=jax.ShapeDtypeStruct((B, C), jnp.float32),
    )(pooled, data, length.reshape(B, 1), table0, W, b.reshape(1, C))


def kernel(data, length, table, W, b):
    B, L = data.shape
    D = table.shape[1]
    tableM = _tc_build_table(table.T)
    tableR = tableM.reshape(-1, D)
    pooled = _sc_pool(data.reshape(B * L), tableR, B, L)
    t0 = lax.slice(table, (0, 0), (1, D))
    return _tc_head(pooled, data, length, t0, W, b)
